# baseline (speedup 1.0000x reference)
import functools
import jax
import jax.numpy as jnp
from jax import lax
from jax.experimental import pallas as pl
from jax.experimental.pallas import tpu as pltpu
from jax.experimental.pallas import tpu_sc as plsc

N = 10000
E = 160000
D = 256
NC = 2
NS = 16
NPAD = 10240
RPT = NPAD // NS
NROW = 10112
DH = 128
CH = 128
NCH = 80
EPAD = NS * NCH * CH
NBUF = 2
EPT_DEG = E // (NC * NS)
DEG_FULL = EPT_DEG // CH
DEG_REM = EPT_DEG - DEG_FULL * CH

_mesh = plsc.VectorSubcoreMesh(core_axis_name="c", subcore_axis_name="s")


@functools.partial(
    pl.kernel,
    out_type=jax.ShapeDtypeStruct((NC, NPAD), jnp.float32),
    mesh=_mesh,
    scratch_types=[
        pltpu.VMEM((RPT,), jnp.float32),
        pltpu.VMEM((CH,), jnp.float32),
        pltpu.VMEM((CH,), jnp.int32),
        pltpu.VMEM((DEG_REM,), jnp.int32),
        pltpu.VMEM_SHARED((NPAD,), jnp.float32),
    ],
)
def _deg_kernel(dst_hbm, out_hbm, zbuf, ones_v, didx_v, didx_r, acc_sh):
    c = lax.axis_index("c")
    s = lax.axis_index("s")
    zero16 = jnp.zeros((16,), jnp.float32)
    one16 = jnp.ones((16,), jnp.float32)

    def _zb(i, _):
        zbuf[pl.ds(i * 16, 16)] = zero16
        return 0
    lax.fori_loop(0, RPT // 16, _zb, 0)
    for j in range(CH // 16):
        ones_v[pl.ds(j * 16, 16)] = one16
    pltpu.sync_copy(zbuf, acc_sh.at[pl.ds(s * RPT, RPT)])
    plsc.subcore_barrier()

    base = (c * NS + s) * EPT_DEG

    def _chunk(i, _):
        off = pl.multiple_of(base + i * CH, 8)
        pltpu.sync_copy(dst_hbm.at[pl.ds(off, CH)], didx_v)
        pltpu.sync_copy(ones_v, acc_sh.at[didx_v], add=True)
        return 0
    lax.fori_loop(0, DEG_FULL, _chunk, 0)
    off = pl.multiple_of(base + DEG_FULL * CH, 8)
    pltpu.sync_copy(dst_hbm.at[pl.ds(off, DEG_REM)], didx_r)
    pltpu.sync_copy(ones_v.at[pl.ds(0, DEG_REM)], acc_sh.at[didx_r], add=True)

    plsc.subcore_barrier()
    pltpu.sync_copy(acc_sh.at[pl.ds(s * RPT, RPT)],
                    out_hbm.at[c, pl.ds(s * RPT, RPT)])


@functools.partial(
    pl.kernel,
    out_type=jax.ShapeDtypeStruct((NC, NROW, DH), jnp.float32),
    mesh=_mesh,
    scratch_types=(
        [pltpu.VMEM((CH, DH), jnp.float32),
         pltpu.VMEM((NCH, CH), jnp.int32),
         pltpu.VMEM((NCH, CH), jnp.int32),
         pltpu.VMEM_SHARED((NROW, DH), jnp.float32),
         pltpu.SemaphoreType.DMA]
    ),
)
def _mp_kernel(tab_hbm, src_hbm, dst_hbm, out_hbm,
               rows_v, gidx_all, didx_all, acc_sh, sem):
    c = lax.axis_index("c")
    s = lax.axis_index("s")
    zero16 = jnp.zeros((16,), jnp.float32)
    rpt = NROW // NS

    def _zr(i, _):
        for j in range(DH // 16):
            rows_v[i, pl.ds(j * 16, 16)] = zero16
        return 0
    lax.fori_loop(0, CH, _zr, 0)
    for k in range(-(-rpt // CH)):
        nr = min(CH, rpt - k * CH)
        pltpu.sync_copy(rows_v.at[pl.ds(0, nr)],
                        acc_sh.at[pl.ds(s * rpt + k * CH, nr)])

    pltpu.sync_copy(src_hbm.at[s], gidx_all)
    pltpu.sync_copy(dst_hbm.at[s], didx_all)

    def _ti(r, _):
        for k in range(CH // 16):
            v = gidx_all[r, pl.ds(k * 16, 16)]
            gidx_all[r, pl.ds(k * 16, 16)] = v + v + c
        return 0
    lax.fori_loop(0, NCH, _ti, 0)
    plsc.subcore_barrier()

    def _chunk(i, _):
        pltpu.async_copy(tab_hbm.at[gidx_all.at[i]], rows_v, sem).wait()
        pltpu.sync_copy(rows_v, acc_sh.at[didx_all.at[i]], add=True)
        return 0
    lax.fori_loop(0, NCH, _chunk, 0)

    plsc.subcore_barrier()
    pltpu.sync_copy(acc_sh.at[pl.ds(s * rpt, rpt)],
                    out_hbm.at[c, pl.ds(s * rpt, rpt)])


def _gru_body(W_ref, wi_ref, wh_ref, bi_ref, bh_ref, out_ref):
    W = W_ref[...]
    gi = lax.dot_general(W, wi_ref[...], (((1,), (1,)), ((), ())),
                         preferred_element_type=jnp.float32) + bi_ref[...]
    gh = lax.dot_general(W, wh_ref[...], (((1,), (1,)), ((), ())),
                         preferred_element_type=jnp.float32) + bh_ref[...]
    r = jax.nn.sigmoid(gi[:, :D] + gh[:, :D])
    z = jax.nn.sigmoid(gi[:, D:2 * D] + gh[:, D:2 * D])
    n = jnp.tanh(gi[:, 2 * D:] + r * gh[:, 2 * D:])
    out_ref[...] = (1.0 - z) * n + z * W


def _gru_call(W, wi, wh, bi, bh):
    return pl.pallas_call(
        _gru_body,
        out_shape=jax.ShapeDtypeStruct((D, D), jnp.float32),
    )(W, wi, wh, bi.reshape(1, 3 * D), bh.reshape(1, 3 * D))


def _dinv_block(degp):
    return lax.rsqrt(degp[0] + degp[1] + 1.0)


def _tc1_body(x_ref, w_ref, degp_ref, out_ref):
    dv = _dinv_block(degp_ref[...])
    h = jnp.dot(x_ref[...], w_ref[...], preferred_element_type=jnp.float32)
    out_ref[...] = dv * h


def _tc1_call(x, Wt1, degp3):
    blk = 256
    grid = (NPAD // blk,)
    return pl.pallas_call(
        _tc1_body,
        grid=grid,
        in_specs=[
            pl.BlockSpec((blk, D), lambda i: (i, 0)),
            pl.BlockSpec((D, D), lambda i: (0, 0)),
            pl.BlockSpec((NC, blk, 1), lambda i: (0, i, 0)),
        ],
        out_specs=pl.BlockSpec((blk, D), lambda i: (i, 0)),
        out_shape=jax.ShapeDtypeStruct((N, D), jnp.float32),
    )(x, Wt1, degp3)


def _tc2_body(S_ref, hp_ref, degp_ref, l0w_ref, l0b_ref, w2_ref, out_ref):
    dv = _dinv_block(degp_ref[...])
    S = S_ref[...]
    hp = hp_ref[...]
    o1 = jnp.concatenate([S[0], S[1]], axis=1) + hp
    a = jax.nn.relu(dv * o1)
    t = lax.dot_general(a, l0w_ref[...], (((1,), (1,)), ((), ())),
                        preferred_element_type=jnp.float32) + l0b_ref[...]
    h2 = jnp.dot(t, w2_ref[...], preferred_element_type=jnp.float32)
    out_ref[...] = dv * h2


def _tc2_call(S1, hp1, degp3, l0w, l0b, Wt2):
    blk = 256
    grid = (NPAD // blk,)
    return pl.pallas_call(
        _tc2_body,
        grid=grid,
        in_specs=[
            pl.BlockSpec((NC, blk, DH), lambda i: (0, i, 0)),
            pl.BlockSpec((blk, D), lambda i: (i, 0)),
            pl.BlockSpec((NC, blk, 1), lambda i: (0, i, 0)),
            pl.BlockSpec((D, D), lambda i: (0, 0)),
            pl.BlockSpec((1, D), lambda i: (0, 0)),
            pl.BlockSpec((D, D), lambda i: (0, 0)),
        ],
        out_specs=pl.BlockSpec((blk, D), lambda i: (i, 0)),
        out_shape=jax.ShapeDtypeStruct((N, D), jnp.float32),
    )(S1, hp1, degp3, l0w, l0b.reshape(1, D), Wt2)


def _tc3_body(S_ref, hp_ref, degp_ref, l1w_ref, l1b_ref, out_ref):
    dv = _dinv_block(degp_ref[...])
    S = S_ref[...]
    o2 = dv * (jnp.concatenate([S[0], S[1]], axis=1) + hp_ref[...])
    y = lax.dot_general(o2, l1w_ref[...], (((1,), (1,)), ((), ())),
                        preferred_element_type=jnp.float32) + l1b_ref[...]
    out_ref[...] = jax.nn.sigmoid(y)


def _tc3_call(S2, hp2, degp3, l1w, l1b):
    blk = 256
    grid = (NPAD // blk,)
    DO = 64
    return pl.pallas_call(
        _tc3_body,
        grid=grid,
        in_specs=[
            pl.BlockSpec((NC, blk, DH), lambda i: (0, i, 0)),
            pl.BlockSpec((blk, D), lambda i: (i, 0)),
            pl.BlockSpec((NC, blk, 1), lambda i: (0, i, 0)),
            pl.BlockSpec((DO, D), lambda i: (0, 0)),
            pl.BlockSpec((1, DO), lambda i: (0, 0)),
        ],
        out_specs=pl.BlockSpec((blk, DO), lambda i: (i, 0)),
        out_shape=jax.ShapeDtypeStruct((N, DO), jnp.float32),
    )(S2, hp2, degp3, l1w, l1b.reshape(1, DO))


def kernel(x, edge_index, weight1, gru1_wi, gru1_wh, gru1_bi, gru1_bh,
           weight2, gru2_wi, gru2_wh, gru2_bi, gru2_bh,
           lin0_w, lin0_b, lin1_w, lin1_b):
    src = edge_index[0].astype(jnp.int32)
    dst = edge_index[1].astype(jnp.int32)
    srcp = jnp.concatenate(
        [src, jnp.zeros((EPAD - E,), jnp.int32)]).reshape(NS, NCH, CH)
    dstp = jnp.concatenate(
        [dst, jnp.full((EPAD - E,), N, jnp.int32)]).reshape(NS, NCH, CH)

    degp = _deg_kernel(dst)
    degp3 = degp.reshape(NC, NPAD, 1)

    Wt1 = _gru_call(weight1, gru1_wi, gru1_wh, gru1_bi, gru1_bh)
    Wt2 = _gru_call(weight2, gru2_wi, gru2_wh, gru2_bi, gru2_bh)

    hp1 = _tc1_call(x, Wt1, degp3)
    S1 = _mp_kernel(hp1.reshape(2 * N, DH), srcp, dstp)
    hp2 = _tc2_call(S1, hp1, degp3, lin0_w, lin0_b, Wt2)
    S2 = _mp_kernel(hp2.reshape(2 * N, DH), srcp, dstp)
    return _tc3_call(S2, hp2, degp3, lin1_w, lin1_b)


# speedup vs baseline: 1.3624x; 1.3624x over previous
import functools
import jax
import jax.numpy as jnp
from jax import lax
from jax.experimental import pallas as pl
from jax.experimental.pallas import tpu as pltpu
from jax.experimental.pallas import tpu_sc as plsc

N = 10000
E = 160000
D = 256
NC = 2
NS = 16
NPAD = 10240
RPT = NPAD // NS
DH = 128
CH = 128
EPT_MP = E // NS
MP_FULL = EPT_MP // CH
MP_REM = EPT_MP - MP_FULL * CH
EPT_DEG = E // (NC * NS)
DEG_FULL = EPT_DEG // CH
DEG_REM = EPT_DEG - DEG_FULL * CH

_mesh = plsc.VectorSubcoreMesh(core_axis_name="c", subcore_axis_name="s")


@functools.partial(
    pl.kernel,
    out_type=jax.ShapeDtypeStruct((NC, NPAD), jnp.float32),
    mesh=_mesh,
    scratch_types=[
        pltpu.VMEM((RPT,), jnp.float32),
        pltpu.VMEM((CH,), jnp.float32),
        pltpu.VMEM((CH,), jnp.int32),
        pltpu.VMEM((DEG_REM,), jnp.int32),
        pltpu.VMEM_SHARED((NPAD,), jnp.float32),
    ],
)
def _deg_kernel(dst_hbm, out_hbm, zbuf, ones_v, didx_v, didx_r, acc_sh):
    c = lax.axis_index("c")
    s = lax.axis_index("s")
    zero16 = jnp.zeros((16,), jnp.float32)
    one16 = jnp.ones((16,), jnp.float32)

    def _zb(i, _):
        zbuf[pl.ds(i * 16, 16)] = zero16
        return 0
    lax.fori_loop(0, RPT // 16, _zb, 0)
    for j in range(CH // 16):
        ones_v[pl.ds(j * 16, 16)] = one16
    pltpu.sync_copy(zbuf, acc_sh.at[pl.ds(s * RPT, RPT)])
    plsc.subcore_barrier()

    base = (c * NS + s) * EPT_DEG

    def _chunk(i, _):
        off = pl.multiple_of(base + i * CH, 8)
        pltpu.sync_copy(dst_hbm.at[pl.ds(off, CH)], didx_v)
        pltpu.sync_copy(ones_v, acc_sh.at[didx_v], add=True)
        return 0
    lax.fori_loop(0, DEG_FULL, _chunk, 0)
    off = pl.multiple_of(base + DEG_FULL * CH, 8)
    pltpu.sync_copy(dst_hbm.at[pl.ds(off, DEG_REM)], didx_r)
    pltpu.sync_copy(ones_v.at[pl.ds(0, DEG_REM)], acc_sh.at[didx_r], add=True)

    plsc.subcore_barrier()
    pltpu.sync_copy(acc_sh.at[pl.ds(s * RPT, RPT)],
                    out_hbm.at[c, pl.ds(s * RPT, RPT)])


@functools.partial(
    pl.kernel,
    out_type=jax.ShapeDtypeStruct((NC, NPAD, DH), jnp.float32),
    mesh=_mesh,
    scratch_types=[
        pltpu.VMEM((CH, DH), jnp.float32),
        pltpu.VMEM((CH,), jnp.int32),
        pltpu.VMEM((CH,), jnp.int32),
        pltpu.VMEM((CH,), jnp.int32),
        pltpu.VMEM((MP_REM, DH), jnp.float32),
        pltpu.VMEM((MP_REM,), jnp.int32),
        pltpu.VMEM((MP_REM,), jnp.int32),
        pltpu.VMEM((MP_REM,), jnp.int32),
        pltpu.VMEM_SHARED((NPAD, DH), jnp.float32),
        pltpu.SemaphoreType.DMA,
    ],
)
def _mp_kernel(tab_hbm, src_hbm, dst_hbm, out_hbm,
               rows_v, sidx_v, didx_v, gidx_v,
               rows_r, sidx_r, didx_r, gidx_r, acc_sh, sem):
    c = lax.axis_index("c")
    s = lax.axis_index("s")
    zero16 = jnp.zeros((16,), jnp.float32)

    def _zr(i, _):
        for j in range(DH // 16):
            rows_v[i, pl.ds(j * 16, 16)] = zero16
        return 0
    lax.fori_loop(0, CH, _zr, 0)
    for k in range(RPT // CH):
        pltpu.sync_copy(rows_v, acc_sh.at[pl.ds(s * RPT + k * CH, CH)])
    plsc.subcore_barrier()

    base = s * EPT_MP

    def _chunk(i, _):
        off = pl.multiple_of(base + i * CH, 8)
        pltpu.sync_copy(src_hbm.at[pl.ds(off, CH)], sidx_v)
        pltpu.sync_copy(dst_hbm.at[pl.ds(off, CH)], didx_v)
        for j in range(CH // 16):
            v = sidx_v[pl.ds(j * 16, 16)]
            gidx_v[pl.ds(j * 16, 16)] = v + v + c
        pltpu.async_copy(tab_hbm.at[gidx_v], rows_v, sem).wait()
        pltpu.sync_copy(rows_v, acc_sh.at[didx_v], add=True)
        return 0
    lax.fori_loop(0, MP_FULL, _chunk, 0)

    off = pl.multiple_of(base + MP_FULL * CH, 8)
    pltpu.sync_copy(src_hbm.at[pl.ds(off, MP_REM)], sidx_r)
    pltpu.sync_copy(dst_hbm.at[pl.ds(off, MP_REM)], didx_r)
    for j in range(MP_REM // 16):
        v = sidx_r[pl.ds(j * 16, 16)]
        gidx_r[pl.ds(j * 16, 16)] = v + v + c
    pltpu.async_copy(tab_hbm.at[gidx_r], rows_r, sem).wait()
    pltpu.sync_copy(rows_r, acc_sh.at[didx_r], add=True)

    plsc.subcore_barrier()
    pltpu.sync_copy(acc_sh.at[pl.ds(s * RPT, RPT)],
                    out_hbm.at[c, pl.ds(s * RPT, RPT)])


def _gru_body(W_ref, wi_ref, wh_ref, bi_ref, bh_ref, out_ref):
    W = W_ref[...]
    gi = lax.dot_general(W, wi_ref[...], (((1,), (1,)), ((), ())),
                         preferred_element_type=jnp.float32) + bi_ref[...]
    gh = lax.dot_general(W, wh_ref[...], (((1,), (1,)), ((), ())),
                         preferred_element_type=jnp.float32) + bh_ref[...]
    r = jax.nn.sigmoid(gi[:, :D] + gh[:, :D])
    z = jax.nn.sigmoid(gi[:, D:2 * D] + gh[:, D:2 * D])
    n = jnp.tanh(gi[:, 2 * D:] + r * gh[:, 2 * D:])
    out_ref[...] = (1.0 - z) * n + z * W


def _gru_call(W, wi, wh, bi, bh):
    return pl.pallas_call(
        _gru_body,
        out_shape=jax.ShapeDtypeStruct((D, D), jnp.float32),
    )(W, wi, wh, bi.reshape(1, 3 * D), bh.reshape(1, 3 * D))


def _dinv_block(degp):
    return lax.rsqrt(degp[0] + degp[1] + 1.0)


def _tc1_body(x_ref, w_ref, degp_ref, out_ref):
    dv = _dinv_block(degp_ref[...])
    h = jnp.dot(x_ref[...], w_ref[...], preferred_element_type=jnp.float32)
    out_ref[...] = dv * h


def _tc1_call(x, Wt1, degp3):
    blk = 256
    grid = (NPAD // blk,)
    return pl.pallas_call(
        _tc1_body,
        grid=grid,
        in_specs=[
            pl.BlockSpec((blk, D), lambda i: (i, 0)),
            pl.BlockSpec((D, D), lambda i: (0, 0)),
            pl.BlockSpec((NC, blk, 1), lambda i: (0, i, 0)),
        ],
        out_specs=pl.BlockSpec((blk, D), lambda i: (i, 0)),
        out_shape=jax.ShapeDtypeStruct((N, D), jnp.float32),
    )(x, Wt1, degp3)


def _tc2_body(S_ref, hp_ref, degp_ref, l0w_ref, l0b_ref, w2_ref, out_ref):
    dv = _dinv_block(degp_ref[...])
    S = S_ref[...]
    hp = hp_ref[...]
    o1 = jnp.concatenate([S[0], S[1]], axis=1) + hp
    a = jax.nn.relu(dv * o1)
    t = lax.dot_general(a, l0w_ref[...], (((1,), (1,)), ((), ())),
                        preferred_element_type=jnp.float32) + l0b_ref[...]
    h2 = jnp.dot(t, w2_ref[...], preferred_element_type=jnp.float32)
    out_ref[...] = dv * h2


def _tc2_call(S1, hp1, degp3, l0w, l0b, Wt2):
    blk = 256
    grid = (NPAD // blk,)
    return pl.pallas_call(
        _tc2_body,
        grid=grid,
        in_specs=[
            pl.BlockSpec((NC, blk, DH), lambda i: (0, i, 0)),
            pl.BlockSpec((blk, D), lambda i: (i, 0)),
            pl.BlockSpec((NC, blk, 1), lambda i: (0, i, 0)),
            pl.BlockSpec((D, D), lambda i: (0, 0)),
            pl.BlockSpec((1, D), lambda i: (0, 0)),
            pl.BlockSpec((D, D), lambda i: (0, 0)),
        ],
        out_specs=pl.BlockSpec((blk, D), lambda i: (i, 0)),
        out_shape=jax.ShapeDtypeStruct((N, D), jnp.float32),
    )(S1, hp1, degp3, l0w, l0b.reshape(1, D), Wt2)


def _tc3_body(S_ref, hp_ref, degp_ref, l1w_ref, l1b_ref, out_ref):
    dv = _dinv_block(degp_ref[...])
    S = S_ref[...]
    o2 = dv * (jnp.concatenate([S[0], S[1]], axis=1) + hp_ref[...])
    y = lax.dot_general(o2, l1w_ref[...], (((1,), (1,)), ((), ())),
                        preferred_element_type=jnp.float32) + l1b_ref[...]
    out_ref[...] = jax.nn.sigmoid(y)


def _tc3_call(S2, hp2, degp3, l1w, l1b):
    blk = 256
    grid = (NPAD // blk,)
    DO = 64
    return pl.pallas_call(
        _tc3_body,
        grid=grid,
        in_specs=[
            pl.BlockSpec((NC, blk, DH), lambda i: (0, i, 0)),
            pl.BlockSpec((blk, D), lambda i: (i, 0)),
            pl.BlockSpec((NC, blk, 1), lambda i: (0, i, 0)),
            pl.BlockSpec((DO, D), lambda i: (0, 0)),
            pl.BlockSpec((1, DO), lambda i: (0, 0)),
        ],
        out_specs=pl.BlockSpec((blk, DO), lambda i: (i, 0)),
        out_shape=jax.ShapeDtypeStruct((N, DO), jnp.float32),
    )(S2, hp2, degp3, l1w, l1b.reshape(1, DO))


def kernel(x, edge_index, weight1, gru1_wi, gru1_wh, gru1_bi, gru1_bh,
           weight2, gru2_wi, gru2_wh, gru2_bi, gru2_bh,
           lin0_w, lin0_b, lin1_w, lin1_b):
    src = edge_index[0].astype(jnp.int32)
    dst = edge_index[1].astype(jnp.int32)

    degp = _deg_kernel(dst)
    degp3 = degp.reshape(NC, NPAD, 1)

    Wt1 = _gru_call(weight1, gru1_wi, gru1_wh, gru1_bi, gru1_bh)
    Wt2 = _gru_call(weight2, gru2_wi, gru2_wh, gru2_bi, gru2_bh)

    hp1 = _tc1_call(x, Wt1, degp3)
    S1 = _mp_kernel(hp1.reshape(2 * N, DH), src, dst)
    hp2 = _tc2_call(S1, hp1, degp3, lin0_w, lin0_b, Wt2)
    S2 = _mp_kernel(hp2.reshape(2 * N, DH), src, dst)
    return _tc3_call(S2, hp2, degp3, lin1_w, lin1_b)


# speedup vs baseline: 1.8279x; 1.3417x over previous
import functools
import jax
import jax.numpy as jnp
from jax import lax
from jax.experimental import pallas as pl
from jax.experimental.pallas import tpu as pltpu
from jax.experimental.pallas import tpu_sc as plsc

N = 10000
E = 160000
D = 256
NC = 2
NS = 16
NPAD = 10240
RPT = NPAD // NS
DH = 128
CH = 128
EPT_MP = E // NS
MP_FULL = EPT_MP // CH
MP_REM = EPT_MP - MP_FULL * CH
EPT_DEG = E // (NC * NS)
DEG_FULL = EPT_DEG // CH
DEG_REM = EPT_DEG - DEG_FULL * CH

_mesh = plsc.VectorSubcoreMesh(core_axis_name="c", subcore_axis_name="s")


@functools.partial(
    pl.kernel,
    out_type=jax.ShapeDtypeStruct((NC, NPAD), jnp.float32),
    mesh=_mesh,
    scratch_types=[
        pltpu.VMEM((RPT,), jnp.float32),
        pltpu.VMEM((CH,), jnp.float32),
        pltpu.VMEM((CH,), jnp.int32),
        pltpu.VMEM((DEG_REM,), jnp.int32),
        pltpu.VMEM_SHARED((NPAD,), jnp.float32),
    ],
)
def _deg_kernel(dst_hbm, out_hbm, zbuf, ones_v, didx_v, didx_r, acc_sh):
    c = lax.axis_index("c")
    s = lax.axis_index("s")
    zero16 = jnp.zeros((16,), jnp.float32)
    one16 = jnp.ones((16,), jnp.float32)

    def _zb(i, _):
        zbuf[pl.ds(i * 16, 16)] = zero16
        return 0
    lax.fori_loop(0, RPT // 16, _zb, 0)
    for j in range(CH // 16):
        ones_v[pl.ds(j * 16, 16)] = one16
    pltpu.sync_copy(zbuf, acc_sh.at[pl.ds(s * RPT, RPT)])
    plsc.subcore_barrier()

    base = (c * NS + s) * EPT_DEG

    def _chunk(i, _):
        off = pl.multiple_of(base + i * CH, 8)
        pltpu.sync_copy(dst_hbm.at[pl.ds(off, CH)], didx_v)
        pltpu.sync_copy(ones_v, acc_sh.at[didx_v], add=True)
        return 0
    lax.fori_loop(0, DEG_FULL, _chunk, 0)
    off = pl.multiple_of(base + DEG_FULL * CH, 8)
    pltpu.sync_copy(dst_hbm.at[pl.ds(off, DEG_REM)], didx_r)
    pltpu.sync_copy(ones_v.at[pl.ds(0, DEG_REM)], acc_sh.at[didx_r], add=True)

    plsc.subcore_barrier()
    pltpu.sync_copy(acc_sh.at[pl.ds(s * RPT, RPT)],
                    out_hbm.at[c, pl.ds(s * RPT, RPT)])


@functools.partial(
    pl.kernel,
    out_type=jax.ShapeDtypeStruct((NC, NPAD, DH), jnp.float32),
    mesh=_mesh,
    scratch_types=[
        pltpu.VMEM((CH, DH), jnp.float32),
        pltpu.VMEM((CH, DH), jnp.float32),
        pltpu.VMEM((CH,), jnp.int32),
        pltpu.VMEM((CH,), jnp.int32),
        pltpu.VMEM((CH,), jnp.int32),
        pltpu.VMEM((CH,), jnp.int32),
        pltpu.VMEM((CH,), jnp.int32),
        pltpu.VMEM((CH,), jnp.int32),
        pltpu.VMEM((MP_REM, DH), jnp.float32),
        pltpu.VMEM((MP_REM,), jnp.int32),
        pltpu.VMEM((MP_REM,), jnp.int32),
        pltpu.VMEM((MP_REM,), jnp.int32),
        pltpu.VMEM_SHARED((NPAD, DH), jnp.float32),
        pltpu.SemaphoreType.DMA,
        pltpu.SemaphoreType.DMA,
    ],
)
def _mp_kernel(tab_hbm, src_hbm, dst_hbm, out_hbm,
               rows_a, rows_b, sidx_a, sidx_b, didx_a, didx_b,
               gidx_a, gidx_b, rows_r, sidx_r, didx_r, gidx_r,
               acc_sh, sem_a, sem_b):
    c = lax.axis_index("c")
    s = lax.axis_index("s")
    zero16 = jnp.zeros((16,), jnp.float32)
    bufs = ((rows_a, sidx_a, didx_a, gidx_a, sem_a),
            (rows_b, sidx_b, didx_b, gidx_b, sem_b))

    def _zr(i, _):
        for j in range(DH // 16):
            rows_a[i, pl.ds(j * 16, 16)] = zero16
        return 0
    lax.fori_loop(0, CH, _zr, 0)
    for k in range(RPT // CH):
        pltpu.sync_copy(rows_a, acc_sh.at[pl.ds(s * RPT + k * CH, CH)])
    plsc.subcore_barrier()

    base = s * EPT_MP
    GRP = 6

    def _load_idx(off, t):
        r, si, di, gi, sm = bufs[t]
        pltpu.sync_copy(src_hbm.at[pl.ds(off, CH)], si)
        pltpu.sync_copy(dst_hbm.at[pl.ds(off, CH)], di)
        for q in range(CH // 16):
            v = si[pl.ds(q * 16, 16)]
            gi[pl.ds(q * 16, 16)] = v + v + c

    def _gather(t):
        r, si, di, gi, sm = bufs[t]
        return pltpu.async_copy(tab_hbm.at[gi], r, sm)

    def _scatter(t):
        r, si, di, gi, sm = bufs[t]
        pltpu.sync_copy(r, acc_sh.at[di], add=True)

    def _outer(g, _):
        base_g = pl.multiple_of(base + g * (GRP * CH), 8)
        _load_idx(base_g, 0)
        d = _gather(0)
        for k in range(GRP):
            t = k % 2
            if k + 1 < GRP:
                _load_idx(base_g + (k + 1) * CH, (k + 1) % 2)
                dn = _gather((k + 1) % 2)
            d.wait()
            _scatter(t)
            if k + 1 < GRP:
                d = dn
        return 0
    lax.fori_loop(0, MP_FULL // GRP, _outer, 0)

    off = pl.multiple_of(base + MP_FULL * CH, 8)
    pltpu.sync_copy(src_hbm.at[pl.ds(off, MP_REM)], sidx_r)
    pltpu.sync_copy(dst_hbm.at[pl.ds(off, MP_REM)], didx_r)
    for j in range(MP_REM // 16):
        v = sidx_r[pl.ds(j * 16, 16)]
        gidx_r[pl.ds(j * 16, 16)] = v + v + c
    pltpu.async_copy(tab_hbm.at[gidx_r], rows_r, sem_a).wait()
    pltpu.sync_copy(rows_r, acc_sh.at[didx_r], add=True)

    plsc.subcore_barrier()
    pltpu.sync_copy(acc_sh.at[pl.ds(s * RPT, RPT)],
                    out_hbm.at[c, pl.ds(s * RPT, RPT)])


def _gru_body(W_ref, wi_ref, wh_ref, bi_ref, bh_ref, out_ref):
    W = W_ref[...]
    gi = lax.dot_general(W, wi_ref[...], (((1,), (1,)), ((), ())),
                         preferred_element_type=jnp.float32) + bi_ref[...]
    gh = lax.dot_general(W, wh_ref[...], (((1,), (1,)), ((), ())),
                         preferred_element_type=jnp.float32) + bh_ref[...]
    r = jax.nn.sigmoid(gi[:, :D] + gh[:, :D])
    z = jax.nn.sigmoid(gi[:, D:2 * D] + gh[:, D:2 * D])
    n = jnp.tanh(gi[:, 2 * D:] + r * gh[:, 2 * D:])
    out_ref[...] = (1.0 - z) * n + z * W


def _gru_call(W, wi, wh, bi, bh):
    return pl.pallas_call(
        _gru_body,
        out_shape=jax.ShapeDtypeStruct((D, D), jnp.float32),
    )(W, wi, wh, bi.reshape(1, 3 * D), bh.reshape(1, 3 * D))


def _dinv_block(degp):
    return lax.rsqrt(degp[0] + degp[1] + 1.0)


def _tc1_body(x_ref, w_ref, degp_ref, out_ref):
    dv = _dinv_block(degp_ref[...])
    h = jnp.dot(x_ref[...], w_ref[...], preferred_element_type=jnp.float32)
    out_ref[...] = dv * h


def _tc1_call(x, Wt1, degp3):
    blk = 256
    grid = (NPAD // blk,)
    return pl.pallas_call(
        _tc1_body,
        grid=grid,
        in_specs=[
            pl.BlockSpec((blk, D), lambda i: (i, 0)),
            pl.BlockSpec((D, D), lambda i: (0, 0)),
            pl.BlockSpec((NC, blk, 1), lambda i: (0, i, 0)),
        ],
        out_specs=pl.BlockSpec((blk, D), lambda i: (i, 0)),
        out_shape=jax.ShapeDtypeStruct((N, D), jnp.float32),
    )(x, Wt1, degp3)


def _tc2_body(S_ref, hp_ref, degp_ref, l0w_ref, l0b_ref, w2_ref, out_ref):
    dv = _dinv_block(degp_ref[...])
    S = S_ref[...]
    hp = hp_ref[...]
    o1 = jnp.concatenate([S[0], S[1]], axis=1) + hp
    a = jax.nn.relu(dv * o1)
    t = lax.dot_general(a, l0w_ref[...], (((1,), (1,)), ((), ())),
                        preferred_element_type=jnp.float32) + l0b_ref[...]
    h2 = jnp.dot(t, w2_ref[...], preferred_element_type=jnp.float32)
    out_ref[...] = dv * h2


def _tc2_call(S1, hp1, degp3, l0w, l0b, Wt2):
    blk = 256
    grid = (NPAD // blk,)
    return pl.pallas_call(
        _tc2_body,
        grid=grid,
        in_specs=[
            pl.BlockSpec((NC, blk, DH), lambda i: (0, i, 0)),
            pl.BlockSpec((blk, D), lambda i: (i, 0)),
            pl.BlockSpec((NC, blk, 1), lambda i: (0, i, 0)),
            pl.BlockSpec((D, D), lambda i: (0, 0)),
            pl.BlockSpec((1, D), lambda i: (0, 0)),
            pl.BlockSpec((D, D), lambda i: (0, 0)),
        ],
        out_specs=pl.BlockSpec((blk, D), lambda i: (i, 0)),
        out_shape=jax.ShapeDtypeStruct((N, D), jnp.float32),
    )(S1, hp1, degp3, l0w, l0b.reshape(1, D), Wt2)


def _tc3_body(S_ref, hp_ref, degp_ref, l1w_ref, l1b_ref, out_ref):
    dv = _dinv_block(degp_ref[...])
    S = S_ref[...]
    o2 = dv * (jnp.concatenate([S[0], S[1]], axis=1) + hp_ref[...])
    y = lax.dot_general(o2, l1w_ref[...], (((1,), (1,)), ((), ())),
                        preferred_element_type=jnp.float32) + l1b_ref[...]
    out_ref[...] = jax.nn.sigmoid(y)


def _tc3_call(S2, hp2, degp3, l1w, l1b):
    blk = 256
    grid = (NPAD // blk,)
    DO = 64
    return pl.pallas_call(
        _tc3_body,
        grid=grid,
        in_specs=[
            pl.BlockSpec((NC, blk, DH), lambda i: (0, i, 0)),
            pl.BlockSpec((blk, D), lambda i: (i, 0)),
            pl.BlockSpec((NC, blk, 1), lambda i: (0, i, 0)),
            pl.BlockSpec((DO, D), lambda i: (0, 0)),
            pl.BlockSpec((1, DO), lambda i: (0, 0)),
        ],
        out_specs=pl.BlockSpec((blk, DO), lambda i: (i, 0)),
        out_shape=jax.ShapeDtypeStruct((N, DO), jnp.float32),
    )(S2, hp2, degp3, l1w, l1b.reshape(1, DO))


def kernel(x, edge_index, weight1, gru1_wi, gru1_wh, gru1_bi, gru1_bh,
           weight2, gru2_wi, gru2_wh, gru2_bi, gru2_bh,
           lin0_w, lin0_b, lin1_w, lin1_b):
    src = edge_index[0].astype(jnp.int32)
    dst = edge_index[1].astype(jnp.int32)

    degp = _deg_kernel(dst)
    degp3 = degp.reshape(NC, NPAD, 1)

    Wt1 = _gru_call(weight1, gru1_wi, gru1_wh, gru1_bi, gru1_bh)
    Wt2 = _gru_call(weight2, gru2_wi, gru2_wh, gru2_bi, gru2_bh)

    hp1 = _tc1_call(x, Wt1, degp3)
    S1 = _mp_kernel(hp1.reshape(2 * N, DH), src, dst)
    hp2 = _tc2_call(S1, hp1, degp3, lin0_w, lin0_b, Wt2)
    S2 = _mp_kernel(hp2.reshape(2 * N, DH), src, dst)
    return _tc3_call(S2, hp2, degp3, lin1_w, lin1_b)


# speedup vs baseline: 1.8682x; 1.0220x over previous
import functools
import jax
import jax.numpy as jnp
from jax import lax
from jax.experimental import pallas as pl
from jax.experimental.pallas import tpu as pltpu
from jax.experimental.pallas import tpu_sc as plsc

N = 10000
E = 160000
D = 256
NC = 2
NS = 16
NPAD = 10240
RPT = NPAD // NS
DH = 128
CH = 128
EPT_MP = E // NS
MP_FULL = EPT_MP // CH
MP_REM = EPT_MP - MP_FULL * CH
EPT_DEG = E // (NC * NS)
DEG_FULL = EPT_DEG // CH
DEG_REM = EPT_DEG - DEG_FULL * CH

_mesh = plsc.VectorSubcoreMesh(core_axis_name="c", subcore_axis_name="s")


@functools.partial(
    pl.kernel,
    out_type=jax.ShapeDtypeStruct((NC, NPAD), jnp.float32),
    mesh=_mesh,
    scratch_types=[
        pltpu.VMEM((RPT,), jnp.float32),
        pltpu.VMEM((CH,), jnp.float32),
        pltpu.VMEM((CH,), jnp.int32),
        pltpu.VMEM((DEG_REM,), jnp.int32),
        pltpu.VMEM_SHARED((NPAD,), jnp.float32),
    ],
)
def _deg_kernel(dst_hbm, out_hbm, zbuf, ones_v, didx_v, didx_r, acc_sh):
    c = lax.axis_index("c")
    s = lax.axis_index("s")
    zero16 = jnp.zeros((16,), jnp.float32)
    one16 = jnp.ones((16,), jnp.float32)

    def _zb(i, _):
        zbuf[pl.ds(i * 16, 16)] = zero16
        return 0
    lax.fori_loop(0, RPT // 16, _zb, 0)
    for j in range(CH // 16):
        ones_v[pl.ds(j * 16, 16)] = one16
    pltpu.sync_copy(zbuf, acc_sh.at[pl.ds(s * RPT, RPT)])
    plsc.subcore_barrier()

    base = (c * NS + s) * EPT_DEG

    def _chunk(i, _):
        off = pl.multiple_of(base + i * CH, 8)
        pltpu.sync_copy(dst_hbm.at[pl.ds(off, CH)], didx_v)
        pltpu.sync_copy(ones_v, acc_sh.at[didx_v], add=True)
        return 0
    lax.fori_loop(0, DEG_FULL, _chunk, 0)
    off = pl.multiple_of(base + DEG_FULL * CH, 8)
    pltpu.sync_copy(dst_hbm.at[pl.ds(off, DEG_REM)], didx_r)
    pltpu.sync_copy(ones_v.at[pl.ds(0, DEG_REM)], acc_sh.at[didx_r], add=True)

    plsc.subcore_barrier()
    pltpu.sync_copy(acc_sh.at[pl.ds(s * RPT, RPT)],
                    out_hbm.at[c, pl.ds(s * RPT, RPT)])


@functools.partial(
    pl.kernel,
    out_type=jax.ShapeDtypeStruct((NC, NPAD, DH), jnp.float32),
    mesh=_mesh,
    scratch_types=[
        pltpu.VMEM((CH, DH), jnp.float32),
        pltpu.VMEM((CH, DH), jnp.float32),
        pltpu.VMEM((CH,), jnp.int32),
        pltpu.VMEM((CH,), jnp.int32),
        pltpu.VMEM((CH,), jnp.int32),
        pltpu.VMEM((CH,), jnp.int32),
        pltpu.VMEM((CH,), jnp.int32),
        pltpu.VMEM((CH,), jnp.int32),
        pltpu.VMEM((MP_REM, DH), jnp.float32),
        pltpu.VMEM((MP_REM,), jnp.int32),
        pltpu.VMEM((MP_REM,), jnp.int32),
        pltpu.VMEM((MP_REM,), jnp.int32),
        pltpu.VMEM_SHARED((NPAD, DH), jnp.float32),
        pltpu.SemaphoreType.DMA,
        pltpu.SemaphoreType.DMA,
    ],
)
def _mp_kernel(tab_hbm, src_hbm, dst_hbm, out_hbm,
               rows_a, rows_b, sidx_a, sidx_b, didx_a, didx_b,
               gidx_a, gidx_b, rows_r, sidx_r, didx_r, gidx_r,
               acc_sh, sem_a, sem_b):
    c = lax.axis_index("c")
    s = lax.axis_index("s")
    zero16 = jnp.zeros((16,), jnp.float32)
    bufs = ((rows_a, sidx_a, didx_a, gidx_a, sem_a),
            (rows_b, sidx_b, didx_b, gidx_b, sem_b))

    def _zr(i, _):
        for j in range(DH // 16):
            rows_a[i, pl.ds(j * 16, 16)] = zero16
        return 0
    lax.fori_loop(0, CH, _zr, 0)
    for k in range(RPT // CH):
        pltpu.sync_copy(rows_a, acc_sh.at[pl.ds(s * RPT + k * CH, CH)])
    plsc.subcore_barrier()

    base = s * EPT_MP
    GRP = 13

    def _load_idx(off, t):
        r, si, di, gi, sm = bufs[t]
        pltpu.sync_copy(src_hbm.at[pl.ds(off, CH)], si)
        pltpu.sync_copy(dst_hbm.at[pl.ds(off, CH)], di)
        for q in range(CH // 16):
            v = si[pl.ds(q * 16, 16)]
            gi[pl.ds(q * 16, 16)] = v + v + c

    def _gather(t):
        r, si, di, gi, sm = bufs[t]
        return pltpu.async_copy(tab_hbm.at[gi], r, sm)

    def _scatter(t):
        r, si, di, gi, sm = bufs[t]
        pltpu.sync_copy(r, acc_sh.at[di], add=True)

    def _outer(g, _):
        base_g = pl.multiple_of(base + g * (GRP * CH), 8)
        _load_idx(base_g, 0)
        d = _gather(0)
        for k in range(GRP):
            t = k % 2
            if k + 1 < GRP:
                _load_idx(base_g + (k + 1) * CH, (k + 1) % 2)
                dn = _gather((k + 1) % 2)
            d.wait()
            _scatter(t)
            if k + 1 < GRP:
                d = dn
        return 0
    lax.fori_loop(0, MP_FULL // GRP, _outer, 0)

    off = pl.multiple_of(base + MP_FULL * CH, 8)
    pltpu.sync_copy(src_hbm.at[pl.ds(off, MP_REM)], sidx_r)
    pltpu.sync_copy(dst_hbm.at[pl.ds(off, MP_REM)], didx_r)
    for j in range(MP_REM // 16):
        v = sidx_r[pl.ds(j * 16, 16)]
        gidx_r[pl.ds(j * 16, 16)] = v + v + c
    pltpu.async_copy(tab_hbm.at[gidx_r], rows_r, sem_a).wait()
    pltpu.sync_copy(rows_r, acc_sh.at[didx_r], add=True)

    plsc.subcore_barrier()
    pltpu.sync_copy(acc_sh.at[pl.ds(s * RPT, RPT)],
                    out_hbm.at[c, pl.ds(s * RPT, RPT)])


def _gru_body(W_ref, wi_ref, wh_ref, bi_ref, bh_ref, out_ref):
    W = W_ref[...]
    gi = lax.dot_general(W, wi_ref[...], (((1,), (1,)), ((), ())),
                         preferred_element_type=jnp.float32) + bi_ref[...]
    gh = lax.dot_general(W, wh_ref[...], (((1,), (1,)), ((), ())),
                         preferred_element_type=jnp.float32) + bh_ref[...]
    r = jax.nn.sigmoid(gi[:, :D] + gh[:, :D])
    z = jax.nn.sigmoid(gi[:, D:2 * D] + gh[:, D:2 * D])
    n = jnp.tanh(gi[:, 2 * D:] + r * gh[:, 2 * D:])
    out_ref[...] = (1.0 - z) * n + z * W


def _gru_call(W, wi, wh, bi, bh):
    return pl.pallas_call(
        _gru_body,
        out_shape=jax.ShapeDtypeStruct((D, D), jnp.float32),
    )(W, wi, wh, bi.reshape(1, 3 * D), bh.reshape(1, 3 * D))


def _dinv_block(degp):
    return lax.rsqrt(degp[0] + degp[1] + 1.0)


def _tc1_body(x_ref, w_ref, degp_ref, out_ref):
    dv = _dinv_block(degp_ref[...])
    h = jnp.dot(x_ref[...], w_ref[...], preferred_element_type=jnp.float32)
    out_ref[...] = dv * h


def _tc1_call(x, Wt1, degp3):
    blk = 256
    grid = (NPAD // blk,)
    return pl.pallas_call(
        _tc1_body,
        grid=grid,
        in_specs=[
            pl.BlockSpec((blk, D), lambda i: (i, 0)),
            pl.BlockSpec((D, D), lambda i: (0, 0)),
            pl.BlockSpec((NC, blk, 1), lambda i: (0, i, 0)),
        ],
        out_specs=pl.BlockSpec((blk, D), lambda i: (i, 0)),
        out_shape=jax.ShapeDtypeStruct((N, D), jnp.float32),
    )(x, Wt1, degp3)


def _tc2_body(S_ref, hp_ref, degp_ref, l0w_ref, l0b_ref, w2_ref, out_ref):
    dv = _dinv_block(degp_ref[...])
    S = S_ref[...]
    hp = hp_ref[...]
    o1 = jnp.concatenate([S[0], S[1]], axis=1) + hp
    a = jax.nn.relu(dv * o1)
    t = lax.dot_general(a, l0w_ref[...], (((1,), (1,)), ((), ())),
                        preferred_element_type=jnp.float32) + l0b_ref[...]
    h2 = jnp.dot(t, w2_ref[...], preferred_element_type=jnp.float32)
    out_ref[...] = dv * h2


def _tc2_call(S1, hp1, degp3, l0w, l0b, Wt2):
    blk = 256
    grid = (NPAD // blk,)
    return pl.pallas_call(
        _tc2_body,
        grid=grid,
        in_specs=[
            pl.BlockSpec((NC, blk, DH), lambda i: (0, i, 0)),
            pl.BlockSpec((blk, D), lambda i: (i, 0)),
            pl.BlockSpec((NC, blk, 1), lambda i: (0, i, 0)),
            pl.BlockSpec((D, D), lambda i: (0, 0)),
            pl.BlockSpec((1, D), lambda i: (0, 0)),
            pl.BlockSpec((D, D), lambda i: (0, 0)),
        ],
        out_specs=pl.BlockSpec((blk, D), lambda i: (i, 0)),
        out_shape=jax.ShapeDtypeStruct((N, D), jnp.float32),
    )(S1, hp1, degp3, l0w, l0b.reshape(1, D), Wt2)


def _tc3_body(S_ref, hp_ref, degp_ref, l1w_ref, l1b_ref, out_ref):
    dv = _dinv_block(degp_ref[...])
    S = S_ref[...]
    o2 = dv * (jnp.concatenate([S[0], S[1]], axis=1) + hp_ref[...])
    y = lax.dot_general(o2, l1w_ref[...], (((1,), (1,)), ((), ())),
                        preferred_element_type=jnp.float32) + l1b_ref[...]
    out_ref[...] = jax.nn.sigmoid(y)


def _tc3_call(S2, hp2, degp3, l1w, l1b):
    blk = 256
    grid = (NPAD // blk,)
    DO = 64
    return pl.pallas_call(
        _tc3_body,
        grid=grid,
        in_specs=[
            pl.BlockSpec((NC, blk, DH), lambda i: (0, i, 0)),
            pl.BlockSpec((blk, D), lambda i: (i, 0)),
            pl.BlockSpec((NC, blk, 1), lambda i: (0, i, 0)),
            pl.BlockSpec((DO, D), lambda i: (0, 0)),
            pl.BlockSpec((1, DO), lambda i: (0, 0)),
        ],
        out_specs=pl.BlockSpec((blk, DO), lambda i: (i, 0)),
        out_shape=jax.ShapeDtypeStruct((N, DO), jnp.float32),
    )(S2, hp2, degp3, l1w, l1b.reshape(1, DO))


def kernel(x, edge_index, weight1, gru1_wi, gru1_wh, gru1_bi, gru1_bh,
           weight2, gru2_wi, gru2_wh, gru2_bi, gru2_bh,
           lin0_w, lin0_b, lin1_w, lin1_b):
    src = edge_index[0].astype(jnp.int32)
    dst = edge_index[1].astype(jnp.int32)

    degp = _deg_kernel(dst)
    degp3 = degp.reshape(NC, NPAD, 1)

    Wt1 = _gru_call(weight1, gru1_wi, gru1_wh, gru1_bi, gru1_bh)
    Wt2 = _gru_call(weight2, gru2_wi, gru2_wh, gru2_bi, gru2_bh)

    hp1 = _tc1_call(x, Wt1, degp3)
    S1 = _mp_kernel(hp1.reshape(2 * N, DH), src, dst)
    hp2 = _tc2_call(S1, hp1, degp3, lin0_w, lin0_b, Wt2)
    S2 = _mp_kernel(hp2.reshape(2 * N, DH), src, dst)
    return _tc3_call(S2, hp2, degp3, lin1_w, lin1_b)
